# baseline (speedup 1.0000x reference)
import functools

import jax
import jax.numpy as jnp
from jax.experimental import pallas as pl
from jax.experimental.pallas import tpu as pltpu
from jax.experimental.pallas import tpu_sc as plsc

TILE_P1 = 4096
TILE_P2 = 2048
GATHER_WINDOW = 128
PACKED_WIDTH = 128


def _sc_gather(table, idx2d, batch):
    mesh = plsc.VectorSubcoreMesh(core_axis_name="core",
                                  subcore_axis_name="subcore")

    @pl.kernel(out_type=jax.ShapeDtypeStruct((batch, PACKED_WIDTH),
                                             table.dtype),
               mesh=mesh)
    def gather_kernel(tab_hbm, idx_hbm, out_hbm):
        def body(idx_vmem, out_vmem):
            pltpu.sync_copy(tab_hbm.at[idx_vmem.at[0]], out_vmem)

        pltpu.emit_pipeline(
            body,
            grid=(batch // GATHER_WINDOW,),
            in_specs=[pl.BlockSpec((1, GATHER_WINDOW), lambda i: (0, i))],
            out_specs=[pl.BlockSpec((GATHER_WINDOW, PACKED_WIDTH),
                                    lambda i: (i, 0))],
            core_axis_name=("core", "subcore"),
            dimension_semantics=(pltpu.PARALLEL,),
        )(idx_hbm, out_hbm)

    return gather_kernel(table, idx2d)


PACK_CHUNK = 16384


def _pack_kernel(et_ref, o_ref):
    x = et_ref[...]
    xt = x.T
    emb_size = x.shape[0]
    pack = PACKED_WIDTH // emb_size
    wr = PACK_CHUNK // pack
    for w in range(pack):
        o_ref[:, w * emb_size:(w + 1) * emb_size] = (
            xt[w * wr:(w + 1) * wr, :])


def _pack_table(et, num_emb, emb_size):
    pack = PACKED_WIDTH // emb_size
    wr = PACK_CHUNK // pack
    num_chunks = pl.cdiv(num_emb, PACK_CHUNK)
    return pl.pallas_call(
        _pack_kernel,
        grid=(num_chunks,),
        in_specs=[pl.BlockSpec((emb_size, PACK_CHUNK), lambda j: (0, j))],
        out_specs=pl.BlockSpec((wr, PACKED_WIDTH), lambda j: (j, 0)),
        out_shape=jax.ShapeDtypeStruct((num_chunks * wr, PACKED_WIDTH),
                                       et.dtype),
        compiler_params=pltpu.CompilerParams(
            dimension_semantics=("arbitrary",)),
    )(et)


def _unpack_kernel(hp_ref, xm_ref, ht_ref, ht2_ref, u_ref, *, emb_size):
    hp = hp_ref[...]
    xm = xm_ref[...]
    pack = PACKED_WIDTH // emb_size
    h = jnp.zeros((hp.shape[0], emb_size), jnp.float32)
    for r in range(pack):
        h = h + jnp.where(xm == r, hp[:, r * emb_size:(r + 1) * emb_size],
                          0.0)
    hb = h.astype(jnp.bfloat16)
    ht_ref[...] = hb.T
    ht2_ref[...] = (h * 1.4426950408889634).astype(jnp.bfloat16).T
    hf = hb.astype(jnp.float32).T
    u_ref[...] = jnp.sum(jnp.maximum(hf, 0.0), axis=0,
                         keepdims=True) * 1.002 + 1e-3


def _logits_tile_t(wt_ref, ht_ref):
    wb = wt_ref[...].astype(jnp.bfloat16)
    hb = ht_ref[...]
    return jax.lax.dot_general(wb, hb, (((1,), (0,)), ((), ())),
                               preferred_element_type=jnp.float32)


SUM_CHUNK = 128


def _sum_exp2_tile(j, lj_ref, u_ref, s_ref, vocab):
    rows = lj_ref.shape[0]
    u2 = u_ref[...] * 1.4426950408889634
    base = vocab - j * rows

    def body(i, acc):
        x = lj_ref[pl.ds(i * SUM_CHUNK, SUM_CHUNK), :]
        row = jax.lax.broadcasted_iota(jnp.int32, (SUM_CHUNK, 1), 0)
        ok = (row + i * SUM_CHUNK) < base
        p = jnp.exp2(jnp.where(ok, x - u2, -1e30))
        for k in range(SUM_CHUNK // 8):
            acc = acc + p[k * 8:(k + 1) * 8, :]
        return acc

    acc = jax.lax.fori_loop(
        0, rows // SUM_CHUNK, body,
        jnp.zeros((8, u2.shape[1]), jnp.float32))

    @pl.when(j == 0)
    def _():
        s_ref[...] = jnp.zeros_like(s_ref)

    s_ref[...] += jnp.sum(acc, axis=0, keepdims=True)


def _pass1_kernel(wt_ref, ht2_ref, u_ref, s_ref, lj_ref, *, vocab):
    j = pl.program_id(0)
    lj_ref[...] = _logits_tile_t(wt_ref, ht2_ref)
    _sum_exp2_tile(j, lj_ref, u_ref, s_ref, vocab)


def _fused_kernel(wt_ref, ht2b_ref, ub_ref, hta_ref, ua_ref, sa_ref,
                  sb_ref, o_ref, lj_ref, *, vocab):
    j = pl.program_id(0)
    lja = _logits_tile_t(wt_ref, hta_ref)
    ca = ua_ref[...] + jnp.log(sa_ref[...] + 1e-30)
    o_ref[...] = lja - ca
    lj_ref[...] = _logits_tile_t(wt_ref, ht2b_ref)
    _sum_exp2_tile(j, lj_ref, ub_ref, sb_ref, vocab)


def _pass2b_kernel(wt_ref, htb_ref, ub_ref, sb_ref, oin_ref, o_ref):
    lj = _logits_tile_t(wt_ref, htb_ref)
    c = ub_ref[...] + jnp.log(sb_ref[...] + 1e-30)
    o_ref[...] = lj - c


def _pass2_kernel(wt_ref, ht_ref, u_ref, s_ref, o_ref):
    lj = _logits_tile_t(wt_ref, ht_ref)
    c = u_ref[...] + jnp.log(s_ref[...] + 1e-30)
    o_ref[...] = lj - c


def _log_softmax_passes(hp, xm, Wt):
    batch = hp.shape[0]
    vocab, emb_size = Wt.shape

    ht, ht2, u = pl.pallas_call(
        functools.partial(_unpack_kernel, emb_size=emb_size),
        out_shape=[jax.ShapeDtypeStruct((emb_size, batch), jnp.bfloat16),
                   jax.ShapeDtypeStruct((emb_size, batch), jnp.bfloat16),
                   jax.ShapeDtypeStruct((1, batch), jnp.float32)],
    )(hp, xm)

    bh = batch // 2
    nt = pl.cdiv(vocab, TILE_P2)
    arb = pltpu.CompilerParams(dimension_semantics=("arbitrary",))
    wt_spec = pl.BlockSpec((TILE_P2, emb_size), lambda j: (j, 0))
    hA = pl.BlockSpec((emb_size, bh), lambda j: (0, 0))
    hB = pl.BlockSpec((emb_size, bh), lambda j: (0, 1))
    rA = pl.BlockSpec((1, bh), lambda j: (0, 0))
    rB = pl.BlockSpec((1, bh), lambda j: (0, 1))
    r0 = pl.BlockSpec((1, bh), lambda j: (0, 0))
    red_type = jax.ShapeDtypeStruct((1, bh), jnp.float32)
    out_type = jax.ShapeDtypeStruct((vocab, batch), jnp.float32)
    scratch = [pltpu.VMEM((TILE_P2, bh), jnp.float32)]

    s_a = pl.pallas_call(
        functools.partial(_pass1_kernel, vocab=vocab),
        grid=(nt,),
        in_specs=[wt_spec, hA, rA],
        out_specs=r0,
        out_shape=red_type,
        scratch_shapes=scratch,
        compiler_params=arb,
    )(Wt, ht2, u)

    s_b, out_half = pl.pallas_call(
        functools.partial(_fused_kernel, vocab=vocab),
        grid=(nt,),
        in_specs=[wt_spec, hB, rB, hA, rA, r0],
        out_specs=[r0, pl.BlockSpec((TILE_P2, bh), lambda j: (j, 0))],
        out_shape=[red_type, out_type],
        scratch_shapes=scratch,
        compiler_params=arb,
    )(Wt, ht2, u, ht, u, s_a)

    out_t = pl.pallas_call(
        _pass2b_kernel,
        grid=(nt,),
        in_specs=[wt_spec, hB, rB, r0,
                  pl.BlockSpec((8, 128), lambda j: (0, 0))],
        out_specs=pl.BlockSpec((TILE_P2, bh), lambda j: (j, 1)),
        out_shape=out_type,
        input_output_aliases={4: 0},
        compiler_params=arb,
    )(Wt, ht, u, s_b, out_half)
    return out_t


def kernel(x, emb, W):
    batch = x.shape[0]
    num_emb, emb_size = emb.shape
    packed = _pack_table(emb.T, num_emb, emb_size)
    xi = x.astype(jnp.int32)
    pack = PACKED_WIDTH // emb_size
    wr = PACK_CHUNK // pack
    xdiv = (wr * (xi // PACK_CHUNK) + xi % wr).reshape(1, batch)
    xm = ((xi % PACK_CHUNK) // wr).reshape(batch, 1)
    hp = _sc_gather(packed, xdiv, batch)
    out_t = _log_softmax_passes(hp, xm, W.T)
    return out_t.T


# speedup vs baseline: 1.0301x; 1.0301x over previous
import functools

import jax
import jax.numpy as jnp
from jax.experimental import pallas as pl
from jax.experimental.pallas import tpu as pltpu
from jax.experimental.pallas import tpu_sc as plsc

TILE_P1 = 4096
TILE_P2 = 4096
GATHER_WINDOW = 128
PACKED_WIDTH = 128


def _sc_gather(table, idx2d, batch):
    mesh = plsc.VectorSubcoreMesh(core_axis_name="core",
                                  subcore_axis_name="subcore")

    @pl.kernel(out_type=jax.ShapeDtypeStruct((batch, PACKED_WIDTH),
                                             table.dtype),
               mesh=mesh)
    def gather_kernel(tab_hbm, idx_hbm, out_hbm):
        def body(idx_vmem, out_vmem):
            pltpu.sync_copy(tab_hbm.at[idx_vmem.at[0]], out_vmem)

        pltpu.emit_pipeline(
            body,
            grid=(batch // GATHER_WINDOW,),
            in_specs=[pl.BlockSpec((1, GATHER_WINDOW), lambda i: (0, i))],
            out_specs=[pl.BlockSpec((GATHER_WINDOW, PACKED_WIDTH),
                                    lambda i: (i, 0))],
            core_axis_name=("core", "subcore"),
            dimension_semantics=(pltpu.PARALLEL,),
        )(idx_hbm, out_hbm)

    return gather_kernel(table, idx2d)


PACK_CHUNK = 16384


def _pack_kernel(et_ref, o_ref):
    x = et_ref[...]
    xt = x.T
    emb_size = x.shape[0]
    pack = PACKED_WIDTH // emb_size
    wr = PACK_CHUNK // pack
    for w in range(pack):
        o_ref[:, w * emb_size:(w + 1) * emb_size] = (
            xt[w * wr:(w + 1) * wr, :])


def _pack_table(et, num_emb, emb_size):
    pack = PACKED_WIDTH // emb_size
    wr = PACK_CHUNK // pack
    num_chunks = pl.cdiv(num_emb, PACK_CHUNK)
    return pl.pallas_call(
        _pack_kernel,
        grid=(num_chunks,),
        in_specs=[pl.BlockSpec((emb_size, PACK_CHUNK), lambda j: (0, j))],
        out_specs=pl.BlockSpec((wr, PACKED_WIDTH), lambda j: (j, 0)),
        out_shape=jax.ShapeDtypeStruct((num_chunks * wr, PACKED_WIDTH),
                                       et.dtype),
        compiler_params=pltpu.CompilerParams(
            dimension_semantics=("arbitrary",)),
    )(et)


def _unpack_kernel(hp_ref, xm_ref, ht_ref, ht2_ref, u_ref, *, emb_size):
    hp = hp_ref[...]
    xm = xm_ref[...]
    pack = PACKED_WIDTH // emb_size
    h = jnp.zeros((hp.shape[0], emb_size), jnp.float32)
    for r in range(pack):
        h = h + jnp.where(xm == r, hp[:, r * emb_size:(r + 1) * emb_size],
                          0.0)
    hb = h.astype(jnp.bfloat16)
    ht_ref[...] = hb.T
    ht2_ref[...] = (h * 1.4426950408889634).astype(jnp.bfloat16).T
    hf = hb.astype(jnp.float32).T
    u_ref[...] = jnp.sum(jnp.maximum(hf, 0.0), axis=0,
                         keepdims=True) * 1.002 + 1e-3


def _logits_tile_t(wt_ref, ht_ref):
    wb = wt_ref[...].astype(jnp.bfloat16)
    hb = ht_ref[...]
    return jax.lax.dot_general(wb, hb, (((1,), (0,)), ((), ())),
                               preferred_element_type=jnp.float32)


SUM_CHUNK = 128


def _pass1_kernel(wt_ref, ht2_ref, u_ref, s_ref, lj_ref, *, vocab):
    j = pl.program_id(0)
    lj_ref[...] = _logits_tile_t(wt_ref, ht2_ref)
    u2 = u_ref[...] * 1.4426950408889634
    base = vocab - j * TILE_P1

    def body(i, acc):
        x = lj_ref[pl.ds(i * SUM_CHUNK, SUM_CHUNK), :]
        row = jax.lax.broadcasted_iota(jnp.int32, (SUM_CHUNK, 1), 0)
        ok = (row + i * SUM_CHUNK) < base
        p = jnp.exp2(jnp.where(ok, x - u2, -1e30))
        for k in range(SUM_CHUNK // 8):
            acc = acc + p[k * 8:(k + 1) * 8, :]
        return acc

    acc = jax.lax.fori_loop(
        0, TILE_P1 // SUM_CHUNK, body,
        jnp.zeros((8, u2.shape[1]), jnp.float32))

    @pl.when(j == 0)
    def _():
        s_ref[...] = jnp.zeros_like(s_ref)

    s_ref[...] += jnp.sum(acc, axis=0, keepdims=True)


def _pass2_kernel(wt_ref, ht_ref, u_ref, s_ref, o_ref):
    lj = _logits_tile_t(wt_ref, ht_ref)
    c = u_ref[...] + jnp.log(s_ref[...] + 1e-30)
    o_ref[...] = lj - c


def _log_softmax_passes(hp, xm, Wt):
    batch = hp.shape[0]
    vocab, emb_size = Wt.shape

    ht, ht2, u = pl.pallas_call(
        functools.partial(_unpack_kernel, emb_size=emb_size),
        out_shape=[jax.ShapeDtypeStruct((emb_size, batch), jnp.bfloat16),
                   jax.ShapeDtypeStruct((emb_size, batch), jnp.bfloat16),
                   jax.ShapeDtypeStruct((1, batch), jnp.float32)],
    )(hp, xm)

    wt1_spec = pl.BlockSpec((TILE_P1, emb_size), lambda j: (j, 0))
    wt2_spec = pl.BlockSpec((TILE_P2, emb_size), lambda j: (j, 0))
    ht_spec = pl.BlockSpec((emb_size, batch), lambda j: (0, 0))
    red_spec = pl.BlockSpec((1, batch), lambda j: (0, 0))
    red_type = jax.ShapeDtypeStruct((1, batch), jnp.float32)

    s = pl.pallas_call(
        functools.partial(_pass1_kernel, vocab=vocab),
        grid=(pl.cdiv(vocab, TILE_P1),),
        in_specs=[wt1_spec, ht_spec, red_spec],
        out_specs=red_spec,
        out_shape=red_type,
        scratch_shapes=[pltpu.VMEM((TILE_P1, batch), jnp.float32)],
        compiler_params=pltpu.CompilerParams(
            dimension_semantics=("arbitrary",)),
    )(Wt, ht2, u)

    out_t = pl.pallas_call(
        _pass2_kernel,
        grid=(pl.cdiv(vocab, TILE_P2),),
        in_specs=[wt2_spec, ht_spec, red_spec, red_spec],
        out_specs=pl.BlockSpec((TILE_P2, batch), lambda j: (j, 0)),
        out_shape=jax.ShapeDtypeStruct((vocab, batch), jnp.float32),
        compiler_params=pltpu.CompilerParams(
            dimension_semantics=("arbitrary",)),
    )(Wt, ht, u, s)
    return out_t


def kernel(x, emb, W):
    batch = x.shape[0]
    num_emb, emb_size = emb.shape
    packed = _pack_table(emb.T, num_emb, emb_size)
    xi = x.astype(jnp.int32)
    pack = PACKED_WIDTH // emb_size
    wr = PACK_CHUNK // pack
    xdiv = (wr * (xi // PACK_CHUNK) + xi % wr).reshape(1, batch)
    xm = ((xi % PACK_CHUNK) // wr).reshape(batch, 1)
    hp = _sc_gather(packed, xdiv, batch)
    out_t = _log_softmax_passes(hp, xm, W.T)
    return out_t.T
